# baseline (speedup 1.0000x reference)
import functools

import jax
import jax.numpy as jnp
from jax import lax
from jax.experimental import pallas as pl
from jax.experimental.pallas import tpu as pltpu
from jax.experimental.pallas import tpu_sc as plsc

_LANES = 16


@functools.lru_cache(maxsize=None)
def _make_sc_kernel(B, T, V, D, chunk):
    info = plsc.get_sparse_core_info()
    NC, NS = info.num_cores, info.num_subcores
    NW = NC * NS
    N = B * T
    n_chunks = N // chunk
    cpt = B // chunk
    cpw = n_chunks // NW
    nbuf = 5
    nsub = D // _LANES
    pspan = min(cpw // cpt + 1, T)

    mesh = plsc.VectorSubcoreMesh(core_axis_name="c", subcore_axis_name="s")

    @functools.partial(
        pl.kernel,
        mesh=mesh,
        out_type=jax.ShapeDtypeStruct((N, D), jnp.float32),
        scratch_types=[
            pltpu.VMEM((cpw * chunk,), jnp.int32),
            pltpu.VMEM((pspan * D,), jnp.float32),
            pltpu.VMEM((chunk,), jnp.int32),
        ]
        + [pltpu.VMEM((chunk, D), jnp.float32) for _ in range(nbuf)]
        + [pltpu.VMEM((chunk,), jnp.int32) for _ in range(nbuf)]
        + [pltpu.SemaphoreType.DMA for _ in range(2 * nbuf)],
    )
    def k(xt_hbm, tok_hbm, pos_hbm, out_hbm, idx_v, pos_v, ramp_v, *rest):
        bufs = rest[:nbuf]
        dsts = rest[nbuf:2 * nbuf]
        gsems = rest[2 * nbuf:3 * nbuf]
        osems = rest[3 * nbuf:]
        w = lax.axis_index("s") * NC + lax.axis_index("c")
        base = w * cpw
        t0 = base // cpt
        pltpu.sync_copy(pos_hbm.at[pl.ds(t0 * D, pspan * D)], pos_v)
        pltpu.sync_copy(xt_hbm.at[pl.ds(base * chunk, cpw * chunk)], idx_v)

        lane = lax.iota(jnp.int32, _LANES)
        for sub in range(chunk // _LANES):
            ramp_v[pl.ds(sub * _LANES, _LANES)] = (lane + sub * _LANES) * T

        def gather(cc, b):
            idx = idx_v.at[pl.ds(cc * chunk, chunk)]
            return pltpu.make_async_copy(tok_hbm.at[idx], bufs[b], gsems[b])

        def out_copy(cc, b):
            return pltpu.make_async_copy(bufs[b], out_hbm.at[dsts[b]],
                                         osems[b])

        def process(cc, b):
            g = base + cc
            t = g // cpt
            base_b = (g % cpt) * chunk
            off = base_b * T + t
            for sub in range(chunk // _LANES):
                sl = pl.ds(sub * _LANES, _LANES)
                dsts[b][sl] = ramp_v[sl] + off
            prow = (t - t0) * D
            pvec = [pos_v[pl.ds(prow + sub * _LANES, _LANES)]
                    for sub in range(nsub)]

            buf = bufs[b]

            def body(r, carry):
                for sub in range(nsub):
                    plsc.addupdate(buf.at[r, pl.ds(sub * _LANES, _LANES)],
                                   pvec[sub])
                return carry

            lax.fori_loop(0, chunk, body, 0)

        gather(0, 0).start()
        gather(1, 1).start()
        gather(2, 2).start()

        def ring_body(i, carry):
            for b in range(nbuf):
                cc = nbuf * i + b
                b2 = (b + 3) % nbuf

                @pl.when(cc + 3 - nbuf >= 0)
                def _():
                    out_copy(cc + 3 - nbuf, b2).wait()

                @pl.when(cc + 3 < cpw)
                def _():
                    gather(cc + 3, b2).start()

                gather(cc, b).wait()
                process(cc, b)
                out_copy(cc, b).start()
            return carry

        lax.fori_loop(0, cpw // nbuf, ring_body, 0)
        for cc in range(cpw - (nbuf - 3), cpw):
            out_copy(cc, cc % nbuf).wait()

    return k


def kernel(x, token_table, pos_table):
    B, T = x.shape
    V, D = token_table.shape
    chunk = 128
    k = _make_sc_kernel(B, T, V, D, chunk)
    xt = x.T.reshape(T * B).astype(jnp.int32)
    out = k(xt, token_table, pos_table.reshape(T * D))
    return out.reshape(B, T, D)
